# baseline (speedup 1.0000x reference)
import functools

import jax
import jax.numpy as jnp
from jax import lax
from jax.experimental import pallas as pl
from jax.experimental.pallas import tpu as pltpu
from jax.experimental.pallas import tpu_sc as plsc

N = 10000
D = 128
NPAD = 10240
NC, NS = 2, 16
NWORK = NC * NS
W = 64
E2 = 320000 + N
WPW = 164
EPW = WPW * W
EPAD = NWORK * EPW
CHUNK = NPAD // NS
NBUF = 2
IBUF = 4

_NEG = -1e30


def _prep1_body(x_ref, w_ref, asv_ref, adv_ref, xl_ref, as_ref, ad_ref, cv_ref):
    x = x_ref[...]
    xl = jnp.dot(x, w_ref[...], preferred_element_type=jnp.float32)
    xl_ref[0:N, :] = xl
    xl_ref[N:NPAD, :] = jnp.zeros((NPAD - N, D), jnp.float32)
    a_s = jnp.sum(xl * asv_ref[...][None, :], axis=1)
    a_d = jnp.sum(xl * adv_ref[...][None, :], axis=1)
    as_ref[0, 0:N] = a_s
    as_ref[0, N:NPAD] = jnp.full((NPAD - N,), _NEG, jnp.float32)
    ad_ref[0, 0:N] = a_d
    ad_ref[0, N:NPAD] = jnp.zeros((NPAD - N,), jnp.float32)
    ub = jnp.max(a_s) + jnp.max(a_d)
    c = jnp.maximum(ub, 0.2 * ub)
    cv_ref[0, :] = jnp.full((16,), c, jnp.float32)


def _prep2_body(acc_ref, den_ref, b1_ref, w_ref, asv_ref, adv_ref,
                xl_ref, as_ref, ad_ref, cv_ref):
    den = den_ref[0, :] + den_ref[1, :] + 1e-16
    h = (acc_ref[0] + acc_ref[1]) / den[:, None] + b1_ref[...][None, :]
    h = jnp.maximum(h, 0.0)
    iota = lax.broadcasted_iota(jnp.int32, (NPAD, 1), 0)
    h = jnp.where(iota < N, h, 0.0)
    xl = jnp.dot(h, w_ref[...], preferred_element_type=jnp.float32)
    xl_ref[...] = xl
    a_s = jnp.sum(xl * asv_ref[...][None, :], axis=1)
    a_d = jnp.sum(xl * adv_ref[...][None, :], axis=1)
    valid = iota[:, 0] < N
    as_ref[0, :] = jnp.where(valid, a_s, _NEG)
    ad_ref[0, :] = jnp.where(valid, a_d, 0.0)
    ub = (jnp.max(jnp.where(valid, a_s, -3e38))
          + jnp.max(jnp.where(valid, a_d, -3e38)))
    c = jnp.maximum(ub, 0.2 * ub)
    cv_ref[0, :] = jnp.full((16,), c, jnp.float32)


def _final_body(acc_ref, den_ref, b2_ref, out_ref):
    den = den_ref[0, :] + den_ref[1, :] + 1e-16
    h = (acc_ref[0] + acc_ref[1]) / den[:, None] + b2_ref[...][None, :]
    out_ref[...] = jnp.tanh(h)


_prep1 = pl.pallas_call(
    _prep1_body,
    out_shape=[
        jax.ShapeDtypeStruct((NPAD, D), jnp.float32),
        jax.ShapeDtypeStruct((1, NPAD), jnp.float32),
        jax.ShapeDtypeStruct((1, NPAD), jnp.float32),
        jax.ShapeDtypeStruct((1, 16), jnp.float32),
    ],
)

_prep2 = pl.pallas_call(
    _prep2_body,
    out_shape=[
        jax.ShapeDtypeStruct((NPAD, D), jnp.float32),
        jax.ShapeDtypeStruct((1, NPAD), jnp.float32),
        jax.ShapeDtypeStruct((1, NPAD), jnp.float32),
        jax.ShapeDtypeStruct((1, 16), jnp.float32),
    ],
)

_final = pl.pallas_call(
    _final_body,
    out_shape=jax.ShapeDtypeStruct((NPAD, D), jnp.float32),
)


_mesh = plsc.VectorSubcoreMesh(core_axis_name="c", subcore_axis_name="s")


@functools.partial(
    pl.kernel,
    out_type=[
        jax.ShapeDtypeStruct((NC, NPAD, D), jnp.float32),
        jax.ShapeDtypeStruct((NC, NPAD), jnp.float32),
    ],
    mesh=_mesh,
    compiler_params=pltpu.CompilerParams(needs_layout_passes=False),
    scratch_types=[
        pltpu.VMEM((NPAD,), jnp.float32),
        pltpu.VMEM((NPAD,), jnp.float32),
        pltpu.VMEM((16,), jnp.float32),
        [pltpu.VMEM((W,), jnp.float32) for _ in range(NBUF)],
        [pltpu.VMEM((W,), jnp.int32) for _ in range(IBUF)],
        [pltpu.VMEM((W,), jnp.int32) for _ in range(IBUF)],
        [pltpu.VMEM((W, D), jnp.float32) for _ in range(NBUF)],
        pltpu.VMEM((16, D), jnp.float32),
        pltpu.VMEM((CHUNK,), jnp.float32),
        pltpu.VMEM_SHARED((NPAD, D), jnp.float32),
        pltpu.VMEM_SHARED((NPAD,), jnp.float32),
        [pltpu.SemaphoreType.DMA for _ in range(NBUF)],
        [pltpu.SemaphoreType.DMA for _ in range(NBUF)],
        [pltpu.SemaphoreType.DMA for _ in range(IBUF)],
    ],
)
def _gat_edge(xl_hbm, src_hbm, dst_hbm, asrc_hbm, adst_hbm, cvec_hbm,
              acc_out, den_out,
              asrc_v, adst_v, cv_v, exw, srcw, dstw, rows, zbuf, zline,
              acc, den, semg, sems, semi):
    c = lax.axis_index("c")
    s = lax.axis_index("s")
    wid = s * NC + c

    pltpu.sync_copy(asrc_hbm, asrc_v)
    pltpu.sync_copy(adst_hbm, adst_v)
    pltpu.sync_copy(cvec_hbm, cv_v)
    cv = cv_v[...]

    zero16 = jnp.zeros((16,), jnp.float32)
    for t in range(16):
        for j in range(D // 16):
            zbuf[t, pl.ds(j * 16, 16)] = zero16

    def _zline(i, carry):
        zline[pl.ds(i * 16, 16)] = zero16
        return carry
    lax.fori_loop(0, CHUNK // 16, _zline, 0)

    def _zacc(i, carry):
        pltpu.sync_copy(zbuf, acc.at[pl.ds(s * CHUNK + i * 16, 16)])
        return carry
    lax.fori_loop(0, CHUNK // 16, _zacc, 0)
    pltpu.sync_copy(zline, den.at[pl.ds(s * CHUNK, CHUNK)])
    plsc.subcore_barrier()

    base = wid * EPW

    def _idx_fetch(w, bi):
        off = base + w * W
        pltpu.async_copy(src_hbm.at[pl.ds(off, W)], srcw[bi], semi[bi])
        pltpu.async_copy(dst_hbm.at[pl.ds(off, W)], dstw[bi], semi[bi])

    def _idx_wait(w, bi):
        off = base + w * W
        pltpu.make_async_copy(src_hbm.at[pl.ds(off, W)], srcw[bi],
                              semi[bi]).wait()
        pltpu.make_async_copy(dst_hbm.at[pl.ds(off, W)], dstw[bi],
                              semi[bi]).wait()

    def _scat_wait(br):
        pltpu.make_async_copy(rows[br], acc.at[dstw[br]], sems[br]).wait()
        pltpu.make_async_copy(exw[br], den.at[dstw[br]], sems[br]).wait()

    for k in range(3):
        _idx_fetch(k, k)
    _idx_wait(0, 0)
    pltpu.async_copy(xl_hbm.at[srcw[0]], rows[0], semg[0])

    def _block(i, carry):
        w0 = i * IBUF
        for b in range(IBUF):
            w = w0 + b
            bi = b
            br = b % NBUF
            bn = (b + 1) % NBUF

            @pl.when(w >= 1)
            def _():
                _scat_wait(bn)

            @pl.when(w + 1 < WPW)
            def _():
                _idx_wait(w + 1, (b + 1) % IBUF)
                pltpu.async_copy(xl_hbm.at[srcw[(b + 1) % IBUF]],
                                 rows[bn], semg[bn])

            @pl.when(w + 3 < WPW)
            def _():
                _idx_fetch(w + 3, (b + 3) % IBUF)

            pltpu.make_async_copy(
                xl_hbm.at[srcw[bi]], rows[br], semg[br]).wait()
            for g in range(W // 16):
                s16 = srcw[bi][pl.ds(g * 16, 16)]
                d16 = dstw[bi][pl.ds(g * 16, 16)]
                av = plsc.load_gather(asrc_v, [s16])
                bv = plsc.load_gather(adst_v, [d16])
                z = av + bv
                alpha = jnp.maximum(z, 0.2 * z)
                ex = jnp.exp(alpha - cv)
                exw[br][pl.ds(g * 16, 16)] = ex
                for r in range(16):
                    bc = jnp.broadcast_to(ex[r], (16,))
                    row = g * 16 + r
                    for j in range(D // 16):
                        rows[br][row, pl.ds(j * 16, 16)] = (
                            rows[br][row, pl.ds(j * 16, 16)] * bc)
            pltpu.async_copy(rows[br], acc.at[dstw[bi]], sems[br], add=True)
            pltpu.async_copy(exw[br], den.at[dstw[bi]], sems[br], add=True)
        return carry
    lax.fori_loop(0, WPW // IBUF, _block, 0)

    _scat_wait((WPW - 1) % NBUF)
    plsc.subcore_barrier()
    pltpu.sync_copy(acc.at[pl.ds(s * CHUNK, CHUNK)],
                    acc_out.at[c, pl.ds(s * CHUNK, CHUNK)])
    pltpu.sync_copy(den.at[pl.ds(s * CHUNK, CHUNK)],
                    den_out.at[c, pl.ds(s * CHUNK, CHUNK)])


def kernel(x, edge_index, W1, att_src1, att_dst1, bias1,
           W2, att_src2, att_dst2, bias2):
    loop = jnp.arange(N, dtype=jnp.int32)
    pad = jnp.full((EPAD - E2,), N, jnp.int32)
    src_e = jnp.concatenate([edge_index[0].astype(jnp.int32), loop, pad])
    dst_e = jnp.concatenate([edge_index[1].astype(jnp.int32), loop, pad])
    xl1, as1, ad1, cv1 = _prep1(x, W1, att_src1, att_dst1)
    acc1, den1 = _gat_edge(xl1, src_e, dst_e,
                           as1.reshape(NPAD), ad1.reshape(NPAD),
                           cv1.reshape(16))
    xl2, as2, ad2, cv2 = _prep2(acc1, den1, bias1, W2, att_src2, att_dst2)
    acc2, den2 = _gat_edge(xl2, src_e, dst_e,
                           as2.reshape(NPAD), ad2.reshape(NPAD),
                           cv2.reshape(16))
    out = _final(acc2, den2, bias2)
    return out[:N]


# speedup vs baseline: 1.1224x; 1.1224x over previous
import functools

import jax
import jax.numpy as jnp
from jax import lax
from jax.experimental import pallas as pl
from jax.experimental.pallas import tpu as pltpu
from jax.experimental.pallas import tpu_sc as plsc

N = 10000
D = 128
NPAD = 10240
NC, NS = 2, 16
NWORK = NC * NS
W = 48
E2 = 320000 + N
WPW = 216
EPW = WPW * W
EPAD = NWORK * EPW
CHUNK = NPAD // NS
NBUF = 3

_NEG = -1e30


def _prep1_body(x_ref, w_ref, asv_ref, adv_ref, xl_ref, as_ref, ad_ref, cv_ref):
    x = x_ref[...]
    xl = jnp.dot(x, w_ref[...], preferred_element_type=jnp.float32)
    xl_ref[0:N, :] = xl
    xl_ref[N:NPAD, :] = jnp.zeros((NPAD - N, D), jnp.float32)
    a_s = jnp.sum(xl * asv_ref[...][None, :], axis=1)
    a_d = jnp.sum(xl * adv_ref[...][None, :], axis=1)
    as_ref[0, 0:N] = a_s
    as_ref[0, N:NPAD] = jnp.full((NPAD - N,), _NEG, jnp.float32)
    ad_ref[0, 0:N] = a_d
    ad_ref[0, N:NPAD] = jnp.zeros((NPAD - N,), jnp.float32)
    ub = jnp.max(a_s) + jnp.max(a_d)
    c = jnp.maximum(ub, 0.2 * ub)
    cv_ref[0, :] = jnp.full((16,), c, jnp.float32)


def _prep2_body(acc_ref, den_ref, b1_ref, w_ref, asv_ref, adv_ref,
                xl_ref, as_ref, ad_ref, cv_ref):
    den = den_ref[0, :] + den_ref[1, :] + 1e-16
    h = (acc_ref[0] + acc_ref[1]) / den[:, None] + b1_ref[...][None, :]
    h = jnp.maximum(h, 0.0)
    iota = lax.broadcasted_iota(jnp.int32, (NPAD, 1), 0)
    h = jnp.where(iota < N, h, 0.0)
    xl = jnp.dot(h, w_ref[...], preferred_element_type=jnp.float32)
    xl_ref[...] = xl
    a_s = jnp.sum(xl * asv_ref[...][None, :], axis=1)
    a_d = jnp.sum(xl * adv_ref[...][None, :], axis=1)
    valid = iota[:, 0] < N
    as_ref[0, :] = jnp.where(valid, a_s, _NEG)
    ad_ref[0, :] = jnp.where(valid, a_d, 0.0)
    ub = (jnp.max(jnp.where(valid, a_s, -3e38))
          + jnp.max(jnp.where(valid, a_d, -3e38)))
    c = jnp.maximum(ub, 0.2 * ub)
    cv_ref[0, :] = jnp.full((16,), c, jnp.float32)


def _final_body(acc_ref, den_ref, b2_ref, out_ref):
    den = den_ref[0, :] + den_ref[1, :] + 1e-16
    h = (acc_ref[0] + acc_ref[1]) / den[:, None] + b2_ref[...][None, :]
    out_ref[...] = jnp.tanh(h)


_prep1 = pl.pallas_call(
    _prep1_body,
    out_shape=[
        jax.ShapeDtypeStruct((NPAD, D), jnp.float32),
        jax.ShapeDtypeStruct((1, NPAD), jnp.float32),
        jax.ShapeDtypeStruct((1, NPAD), jnp.float32),
        jax.ShapeDtypeStruct((1, 16), jnp.float32),
    ],
)

_prep2 = pl.pallas_call(
    _prep2_body,
    out_shape=[
        jax.ShapeDtypeStruct((NPAD, D), jnp.float32),
        jax.ShapeDtypeStruct((1, NPAD), jnp.float32),
        jax.ShapeDtypeStruct((1, NPAD), jnp.float32),
        jax.ShapeDtypeStruct((1, 16), jnp.float32),
    ],
)

_final = pl.pallas_call(
    _final_body,
    out_shape=jax.ShapeDtypeStruct((NPAD, D), jnp.float32),
)


_mesh = plsc.VectorSubcoreMesh(core_axis_name="c", subcore_axis_name="s")


@functools.partial(
    pl.kernel,
    out_type=[
        jax.ShapeDtypeStruct((NC, NPAD, D), jnp.float32),
        jax.ShapeDtypeStruct((NC, NPAD), jnp.float32),
    ],
    mesh=_mesh,
    compiler_params=pltpu.CompilerParams(needs_layout_passes=False),
    scratch_types=[
        pltpu.VMEM((NPAD,), jnp.float32),
        pltpu.VMEM((NPAD,), jnp.float32),
        pltpu.VMEM((16,), jnp.float32),
        [pltpu.VMEM((W,), jnp.float32) for _ in range(NBUF)],
        [pltpu.VMEM((W,), jnp.int32) for _ in range(NBUF)],
        [pltpu.VMEM((W,), jnp.int32) for _ in range(NBUF)],
        [pltpu.VMEM((W, D), jnp.float32) for _ in range(NBUF)],
        pltpu.VMEM((16, D), jnp.float32),
        pltpu.VMEM((CHUNK,), jnp.float32),
        pltpu.VMEM_SHARED((NPAD, D), jnp.float32),
        pltpu.VMEM_SHARED((NPAD,), jnp.float32),
        [pltpu.SemaphoreType.DMA for _ in range(NBUF)],
        [pltpu.SemaphoreType.DMA for _ in range(NBUF)],
    ],
)
def _gat_edge(xl_hbm, src_hbm, dst_hbm, asrc_hbm, adst_hbm, cvec_hbm,
              acc_out, den_out,
              asrc_v, adst_v, cv_v, exw, srcw, dstw, rows,
              zbuf, zline, acc, den, semg, sems):
    c = lax.axis_index("c")
    s = lax.axis_index("s")
    wid = s * NC + c

    pltpu.sync_copy(asrc_hbm, asrc_v)
    pltpu.sync_copy(adst_hbm, adst_v)
    pltpu.sync_copy(cvec_hbm, cv_v)
    cv = cv_v[...]

    zero16 = jnp.zeros((16,), jnp.float32)
    for t in range(16):
        for j in range(D // 16):
            zbuf[t, pl.ds(j * 16, 16)] = zero16

    def _zline(i, carry):
        zline[pl.ds(i * 16, 16)] = zero16
        return carry
    lax.fori_loop(0, CHUNK // 16, _zline, 0)

    def _zacc(i, carry):
        pltpu.sync_copy(zbuf, acc.at[pl.ds(s * CHUNK + i * 16, 16)])
        return carry
    lax.fori_loop(0, CHUNK // 16, _zacc, 0)
    pltpu.sync_copy(zline, den.at[pl.ds(s * CHUNK, CHUNK)])
    plsc.subcore_barrier()

    base = wid * EPW

    def _fetch(b, w):
        off = base + w * W
        pltpu.sync_copy(src_hbm.at[pl.ds(off, W)], srcw[b])
        pltpu.sync_copy(dst_hbm.at[pl.ds(off, W)], dstw[b])
        pltpu.async_copy(xl_hbm.at[srcw[b]], rows[b], semg[b])

    for b in range(2):
        _fetch(b, b)

    def _block(i, carry):
        w0 = i * NBUF
        for b in range(NBUF):
            w = w0 + b
            pltpu.make_async_copy(
                xl_hbm.at[srcw[b]], rows[b], semg[b]).wait()
            for g in range(W // 16):
                s16 = srcw[b][pl.ds(g * 16, 16)]
                d16 = dstw[b][pl.ds(g * 16, 16)]
                av = plsc.load_gather(asrc_v, [s16])
                bv = plsc.load_gather(adst_v, [d16])
                z = av + bv
                alpha = jnp.maximum(z, 0.2 * z)
                ex = jnp.exp(alpha - cv)
                exw[b][pl.ds(g * 16, 16)] = ex
                for r in range(16):
                    bc = jnp.broadcast_to(ex[r], (16,))
                    row = g * 16 + r
                    for j in range(D // 16):
                        rows[b][row, pl.ds(j * 16, 16)] = (
                            rows[b][row, pl.ds(j * 16, 16)] * bc)
            pltpu.async_copy(rows[b], acc.at[dstw[b]], sems[b], add=True)
            pltpu.async_copy(exw[b], den.at[dstw[b]], sems[b], add=True)

            bp = (b + 2) % NBUF

            @pl.when(jnp.logical_and(w >= 1, w + 2 < WPW))
            def _():
                pltpu.make_async_copy(
                    rows[bp], acc.at[dstw[bp]], sems[bp]).wait()
                pltpu.make_async_copy(
                    exw[bp], den.at[dstw[bp]], sems[bp]).wait()

            @pl.when(w + 2 < WPW)
            def _():
                _fetch(bp, w + 2)
        return carry
    lax.fori_loop(0, WPW // NBUF, _block, 0)

    for b in range(NBUF):
        pltpu.make_async_copy(rows[b], acc.at[dstw[b]], sems[b]).wait()
        pltpu.make_async_copy(exw[b], den.at[dstw[b]], sems[b]).wait()
    plsc.subcore_barrier()
    pltpu.sync_copy(acc.at[pl.ds(s * CHUNK, CHUNK)],
                    acc_out.at[c, pl.ds(s * CHUNK, CHUNK)])
    pltpu.sync_copy(den.at[pl.ds(s * CHUNK, CHUNK)],
                    den_out.at[c, pl.ds(s * CHUNK, CHUNK)])


def kernel(x, edge_index, W1, att_src1, att_dst1, bias1,
           W2, att_src2, att_dst2, bias2):
    loop = jnp.arange(N, dtype=jnp.int32)
    pad = jnp.full((EPAD - E2,), N, jnp.int32)
    src_e = jnp.concatenate([edge_index[0].astype(jnp.int32), loop, pad])
    dst_e = jnp.concatenate([edge_index[1].astype(jnp.int32), loop, pad])
    xl1, as1, ad1, cv1 = _prep1(x, W1, att_src1, att_dst1)
    acc1, den1 = _gat_edge(xl1, src_e, dst_e,
                           as1.reshape(NPAD), ad1.reshape(NPAD),
                           cv1.reshape(16))
    xl2, as2, ad2, cv2 = _prep2(acc1, den1, bias1, W2, att_src2, att_dst2)
    acc2, den2 = _gat_edge(xl2, src_e, dst_e,
                           as2.reshape(NPAD), ad2.reshape(NPAD),
                           cv2.reshape(16))
    out = _final(acc2, den2, bias2)
    return out[:N]


# speedup vs baseline: 1.2141x; 1.0818x over previous
import functools

import jax
import jax.numpy as jnp
from jax import lax
from jax.experimental import pallas as pl
from jax.experimental.pallas import tpu as pltpu
from jax.experimental.pallas import tpu_sc as plsc

N = 10000
D = 128
NPAD = 10240
NC, NS = 2, 16
NWORK = NC * NS
W = 48
E2 = 320000 + N
WPW = 216
EPW = WPW * W
EPAD = NWORK * EPW
CHUNK = NPAD // NS
NBUF = 3

_NEG = -1e30


def _prep1_body(x_ref, w_ref, asv_ref, adv_ref, xl_ref, as_ref, ad_ref, cv_ref):
    x = x_ref[...]
    xl = jnp.dot(x, w_ref[...], preferred_element_type=jnp.float32)
    xl_ref[0:N, :] = xl
    xl_ref[N:NPAD, :] = jnp.zeros((NPAD - N, D), jnp.float32)
    a_s = jnp.sum(xl * asv_ref[...][None, :], axis=1)
    a_d = jnp.sum(xl * adv_ref[...][None, :], axis=1)
    as_ref[0, 0:N] = a_s
    as_ref[0, N:NPAD] = jnp.full((NPAD - N,), _NEG, jnp.float32)
    ad_ref[0, 0:N] = a_d
    ad_ref[0, N:NPAD] = jnp.zeros((NPAD - N,), jnp.float32)
    ub = jnp.max(a_s) + jnp.max(a_d)
    c = jnp.maximum(ub, 0.2 * ub)
    cv_ref[0, :] = jnp.full((16,), c, jnp.float32)


def _prep2_body(acc_ref, den_ref, b1_ref, w_ref, asv_ref, adv_ref,
                xl_ref, as_ref, ad_ref, cv_ref):
    den = den_ref[0, :] + den_ref[1, :] + 1e-16
    h = (acc_ref[0] + acc_ref[1]) / den[:, None] + b1_ref[...][None, :]
    h = jnp.maximum(h, 0.0)
    iota = lax.broadcasted_iota(jnp.int32, (NPAD, 1), 0)
    h = jnp.where(iota < N, h, 0.0)
    xl = jnp.dot(h, w_ref[...], preferred_element_type=jnp.float32)
    xl_ref[...] = xl
    a_s = jnp.sum(xl * asv_ref[...][None, :], axis=1)
    a_d = jnp.sum(xl * adv_ref[...][None, :], axis=1)
    valid = iota[:, 0] < N
    as_ref[0, :] = jnp.where(valid, a_s, _NEG)
    ad_ref[0, :] = jnp.where(valid, a_d, 0.0)
    ub = (jnp.max(jnp.where(valid, a_s, -3e38))
          + jnp.max(jnp.where(valid, a_d, -3e38)))
    c = jnp.maximum(ub, 0.2 * ub)
    cv_ref[0, :] = jnp.full((16,), c, jnp.float32)


def _final_body(acc_ref, den_ref, b2_ref, out_ref):
    den = den_ref[0, :] + den_ref[1, :] + 1e-16
    h = (acc_ref[0] + acc_ref[1]) / den[:, None] + b2_ref[...][None, :]
    out_ref[...] = jnp.tanh(h)


_prep1 = pl.pallas_call(
    _prep1_body,
    out_shape=[
        jax.ShapeDtypeStruct((NPAD, D), jnp.float32),
        jax.ShapeDtypeStruct((1, NPAD), jnp.float32),
        jax.ShapeDtypeStruct((1, NPAD), jnp.float32),
        jax.ShapeDtypeStruct((1, 16), jnp.float32),
    ],
)

_prep2 = pl.pallas_call(
    _prep2_body,
    out_shape=[
        jax.ShapeDtypeStruct((NPAD, D), jnp.float32),
        jax.ShapeDtypeStruct((1, NPAD), jnp.float32),
        jax.ShapeDtypeStruct((1, NPAD), jnp.float32),
        jax.ShapeDtypeStruct((1, 16), jnp.float32),
    ],
)

_final = pl.pallas_call(
    _final_body,
    out_shape=jax.ShapeDtypeStruct((NPAD, D), jnp.float32),
)


_mesh = plsc.VectorSubcoreMesh(core_axis_name="c", subcore_axis_name="s")


@functools.partial(
    pl.kernel,
    out_type=[
        jax.ShapeDtypeStruct((NC, NPAD, D), jnp.float32),
        jax.ShapeDtypeStruct((NC, NPAD), jnp.float32),
    ],
    mesh=_mesh,
    compiler_params=pltpu.CompilerParams(needs_layout_passes=False),
    scratch_types=[
        pltpu.VMEM((NPAD,), jnp.float32),
        pltpu.VMEM((NPAD,), jnp.float32),
        pltpu.VMEM((16,), jnp.float32),
        [pltpu.VMEM((W,), jnp.float32) for _ in range(NBUF)],
        [pltpu.VMEM((W,), jnp.int32) for _ in range(NBUF)],
        [pltpu.VMEM((W,), jnp.int32) for _ in range(NBUF)],
        [pltpu.VMEM((W, D), jnp.float32) for _ in range(NBUF)],
        pltpu.VMEM((16, D), jnp.float32),
        pltpu.VMEM((CHUNK,), jnp.float32),
        pltpu.VMEM_SHARED((NPAD, D), jnp.float32),
        pltpu.VMEM_SHARED((NPAD,), jnp.float32),
        [pltpu.SemaphoreType.DMA for _ in range(NBUF)],
        [pltpu.SemaphoreType.DMA for _ in range(NBUF)],
        [pltpu.SemaphoreType.DMA for _ in range(NBUF)],
    ],
)
def _gat_edge(xl_hbm, src_hbm, dst_hbm, asrc_hbm, adst_hbm, cvec_hbm,
              acc_out, den_out,
              asrc_v, adst_v, cv_v, exw, srcw, dstw, rows,
              zbuf, zline, acc, den, semg, sems, semi):
    c = lax.axis_index("c")
    s = lax.axis_index("s")
    wid = s * NC + c

    pltpu.sync_copy(asrc_hbm, asrc_v)
    pltpu.sync_copy(adst_hbm, adst_v)
    pltpu.sync_copy(cvec_hbm, cv_v)
    cv = cv_v[...]

    zero16 = jnp.zeros((16,), jnp.float32)
    for t in range(16):
        for j in range(D // 16):
            zbuf[t, pl.ds(j * 16, 16)] = zero16

    def _zline(i, carry):
        zline[pl.ds(i * 16, 16)] = zero16
        return carry
    lax.fori_loop(0, CHUNK // 16, _zline, 0)

    def _zacc(i, carry):
        pltpu.sync_copy(zbuf, acc.at[pl.ds(s * CHUNK + i * 16, 16)])
        return carry
    lax.fori_loop(0, CHUNK // 16, _zacc, 0)
    pltpu.sync_copy(zline, den.at[pl.ds(s * CHUNK, CHUNK)])
    plsc.subcore_barrier()

    base = wid * EPW

    def _idx_fetch(b, w):
        off = base + w * W
        pltpu.async_copy(src_hbm.at[pl.ds(off, W)], srcw[b], semi[b])
        pltpu.async_copy(dst_hbm.at[pl.ds(off, W)], dstw[b], semi[b])

    def _idx_wait(b, w):
        off = base + w * W
        pltpu.make_async_copy(src_hbm.at[pl.ds(off, W)], srcw[b],
                              semi[b]).wait()
        pltpu.make_async_copy(dst_hbm.at[pl.ds(off, W)], dstw[b],
                              semi[b]).wait()

    for b in range(2):
        _idx_fetch(b, b)
    _idx_wait(0, 0)
    pltpu.async_copy(xl_hbm.at[srcw[0]], rows[0], semg[0])

    def _block(i, carry):
        w0 = i * NBUF
        for b in range(NBUF):
            w = w0 + b
            bn = (b + 1) % NBUF
            pltpu.make_async_copy(
                xl_hbm.at[srcw[b]], rows[b], semg[b]).wait()

            @pl.when(w + 1 < WPW)
            def _():
                _idx_wait(bn, w + 1)
                pltpu.async_copy(xl_hbm.at[srcw[bn]], rows[bn], semg[bn])

            for g in range(W // 16):
                s16 = srcw[b][pl.ds(g * 16, 16)]
                d16 = dstw[b][pl.ds(g * 16, 16)]
                av = plsc.load_gather(asrc_v, [s16])
                bv = plsc.load_gather(adst_v, [d16])
                z = av + bv
                alpha = jnp.maximum(z, 0.2 * z)
                ex = jnp.exp(alpha - cv)
                exw[b][pl.ds(g * 16, 16)] = ex
                for r in range(16):
                    bc = jnp.broadcast_to(ex[r], (16,))
                    row = g * 16 + r
                    for j in range(D // 16):
                        rows[b][row, pl.ds(j * 16, 16)] = (
                            rows[b][row, pl.ds(j * 16, 16)] * bc)
            pltpu.async_copy(rows[b], acc.at[dstw[b]], sems[b], add=True)
            pltpu.async_copy(exw[b], den.at[dstw[b]], sems[b], add=True)

            bp = (b + 2) % NBUF

            @pl.when(jnp.logical_and(w >= 1, w + 2 < WPW))
            def _():
                pltpu.make_async_copy(
                    rows[bp], acc.at[dstw[bp]], sems[bp]).wait()
                pltpu.make_async_copy(
                    exw[bp], den.at[dstw[bp]], sems[bp]).wait()

            @pl.when(w + 2 < WPW)
            def _():
                _idx_fetch(bp, w + 2)
        return carry
    lax.fori_loop(0, WPW // NBUF, _block, 0)

    for b in range(NBUF):
        pltpu.make_async_copy(rows[b], acc.at[dstw[b]], sems[b]).wait()
        pltpu.make_async_copy(exw[b], den.at[dstw[b]], sems[b]).wait()
    plsc.subcore_barrier()
    pltpu.sync_copy(acc.at[pl.ds(s * CHUNK, CHUNK)],
                    acc_out.at[c, pl.ds(s * CHUNK, CHUNK)])
    pltpu.sync_copy(den.at[pl.ds(s * CHUNK, CHUNK)],
                    den_out.at[c, pl.ds(s * CHUNK, CHUNK)])


def kernel(x, edge_index, W1, att_src1, att_dst1, bias1,
           W2, att_src2, att_dst2, bias2):
    loop = jnp.arange(N, dtype=jnp.int32)
    pad = jnp.full((EPAD - E2,), N, jnp.int32)
    src_e = jnp.concatenate([edge_index[0].astype(jnp.int32), loop, pad])
    dst_e = jnp.concatenate([edge_index[1].astype(jnp.int32), loop, pad])
    xl1, as1, ad1, cv1 = _prep1(x, W1, att_src1, att_dst1)
    acc1, den1 = _gat_edge(xl1, src_e, dst_e,
                           as1.reshape(NPAD), ad1.reshape(NPAD),
                           cv1.reshape(16))
    xl2, as2, ad2, cv2 = _prep2(acc1, den1, bias1, W2, att_src2, att_dst2)
    acc2, den2 = _gat_edge(xl2, src_e, dst_e,
                           as2.reshape(NPAD), ad2.reshape(NPAD),
                           cv2.reshape(16))
    out = _final(acc2, den2, bias2)
    return out[:N]


# speedup vs baseline: 1.2220x; 1.0065x over previous
import functools

import jax
import jax.numpy as jnp
from jax import lax
from jax.experimental import pallas as pl
from jax.experimental.pallas import tpu as pltpu
from jax.experimental.pallas import tpu_sc as plsc

N = 10000
D = 128
NPAD = 10240
NC, NS = 2, 16
NWORK = NC * NS
W = 48
E2 = 320000 + N
WPW = 216
EPW = WPW * W
EPAD = NWORK * EPW
CHUNK = NPAD // NS
NBUF = 3

_NEG = -1e30


def _prep1_body(x_ref, w_ref, asv_ref, adv_ref, xl_ref, as_ref, ad_ref, cv_ref):
    x = x_ref[...]
    xl = jnp.dot(x, w_ref[...], preferred_element_type=jnp.float32)
    xl_ref[0:N, :] = xl
    xl_ref[N:NPAD, :] = jnp.zeros((NPAD - N, D), jnp.float32)
    a_s = jnp.sum(xl * asv_ref[...][None, :], axis=1)
    a_d = jnp.sum(xl * adv_ref[...][None, :], axis=1)
    as_ref[0, 0:N] = a_s
    as_ref[0, N:NPAD] = jnp.full((NPAD - N,), _NEG, jnp.float32)
    ad_ref[0, 0:N] = a_d
    ad_ref[0, N:NPAD] = jnp.zeros((NPAD - N,), jnp.float32)
    ub = jnp.max(a_s) + jnp.max(a_d)
    c = jnp.maximum(ub, 0.2 * ub)
    cv_ref[0, :] = jnp.full((16,), c, jnp.float32)


def _prep2_body(acc_ref, den_ref, b1_ref, w_ref, asv_ref, adv_ref,
                xl_ref, as_ref, ad_ref, cv_ref):
    den = den_ref[0, :] + den_ref[1, :] + 1e-16
    h = (acc_ref[0] + acc_ref[1]) / den[:, None] + b1_ref[...][None, :]
    h = jnp.maximum(h, 0.0)
    iota = lax.broadcasted_iota(jnp.int32, (NPAD, 1), 0)
    h = jnp.where(iota < N, h, 0.0)
    xl = jnp.dot(h, w_ref[...], preferred_element_type=jnp.float32)
    xl_ref[...] = xl
    a_s = jnp.sum(xl * asv_ref[...][None, :], axis=1)
    a_d = jnp.sum(xl * adv_ref[...][None, :], axis=1)
    valid = iota[:, 0] < N
    as_ref[0, :] = jnp.where(valid, a_s, _NEG)
    ad_ref[0, :] = jnp.where(valid, a_d, 0.0)
    ub = (jnp.max(jnp.where(valid, a_s, -3e38))
          + jnp.max(jnp.where(valid, a_d, -3e38)))
    c = jnp.maximum(ub, 0.2 * ub)
    cv_ref[0, :] = jnp.full((16,), c, jnp.float32)


def _final_body(acc_ref, den_ref, b2_ref, out_ref):
    den = den_ref[0, :] + den_ref[1, :] + 1e-16
    h = (acc_ref[0] + acc_ref[1]) / den[:, None] + b2_ref[...][None, :]
    out_ref[...] = jnp.tanh(h)


_prep1 = pl.pallas_call(
    _prep1_body,
    out_shape=[
        jax.ShapeDtypeStruct((NPAD, D), jnp.float32),
        jax.ShapeDtypeStruct((1, NPAD), jnp.float32),
        jax.ShapeDtypeStruct((1, NPAD), jnp.float32),
        jax.ShapeDtypeStruct((1, 16), jnp.float32),
    ],
)

_prep2 = pl.pallas_call(
    _prep2_body,
    out_shape=[
        jax.ShapeDtypeStruct((NPAD, D), jnp.float32),
        jax.ShapeDtypeStruct((1, NPAD), jnp.float32),
        jax.ShapeDtypeStruct((1, NPAD), jnp.float32),
        jax.ShapeDtypeStruct((1, 16), jnp.float32),
    ],
)

_final = pl.pallas_call(
    _final_body,
    out_shape=jax.ShapeDtypeStruct((NPAD, D), jnp.float32),
)


_mesh = plsc.VectorSubcoreMesh(core_axis_name="c", subcore_axis_name="s")


@functools.partial(
    pl.kernel,
    out_type=[
        jax.ShapeDtypeStruct((NC, NPAD, D), jnp.float32),
        jax.ShapeDtypeStruct((NC, NPAD), jnp.float32),
    ],
    mesh=_mesh,
    compiler_params=pltpu.CompilerParams(needs_layout_passes=False),
    scratch_types=[
        pltpu.VMEM((NPAD,), jnp.float32),
        pltpu.VMEM((NPAD,), jnp.float32),
        pltpu.VMEM((16,), jnp.float32),
        [pltpu.VMEM((W,), jnp.float32) for _ in range(NBUF)],
        [pltpu.VMEM((W,), jnp.int32) for _ in range(NBUF)],
        [pltpu.VMEM((W,), jnp.int32) for _ in range(NBUF)],
        [pltpu.VMEM((W, D), jnp.float32) for _ in range(NBUF)],
        pltpu.VMEM((16, D), jnp.float32),
        pltpu.VMEM((CHUNK,), jnp.float32),
        pltpu.VMEM_SHARED((NPAD, D), jnp.float32),
        pltpu.VMEM_SHARED((NPAD,), jnp.float32),
        [pltpu.SemaphoreType.DMA for _ in range(NBUF)],
        [pltpu.SemaphoreType.DMA for _ in range(NBUF)],
        [pltpu.SemaphoreType.DMA for _ in range(NBUF)],
    ],
)
def _gat_edge(xl_hbm, src_hbm, dst_hbm, asrc_hbm, adst_hbm, cvec_hbm,
              acc_out, den_out,
              asrc_v, adst_v, cv_v, exw, srcw, dstw, rows,
              zbuf, zline, acc, den, semg, sems, semi):
    c = lax.axis_index("c")
    s = lax.axis_index("s")
    wid = s * NC + c

    pltpu.sync_copy(asrc_hbm, asrc_v)
    pltpu.sync_copy(adst_hbm, adst_v)
    pltpu.sync_copy(cvec_hbm, cv_v)
    cv = cv_v[...]

    zero16 = jnp.zeros((16,), jnp.float32)
    for t in range(16):
        for j in range(D // 16):
            zbuf[t, pl.ds(j * 16, 16)] = zero16

    def _zline(i, carry):
        zline[pl.ds(i * 16, 16)] = zero16
        return carry
    lax.fori_loop(0, CHUNK // 16, _zline, 0)

    def _zacc(i, carry):
        pltpu.async_copy(zbuf, acc.at[pl.ds(s * CHUNK + i * 16, 16)],
                         semg[0])
        return carry
    lax.fori_loop(0, CHUNK // 16, _zacc, 0)
    pltpu.sync_copy(zline, den.at[pl.ds(s * CHUNK, CHUNK)])

    def _zwait(i, carry):
        pltpu.make_async_copy(
            zbuf, acc.at[pl.ds(s * CHUNK + i * 16, 16)], semg[0]).wait()
        return carry
    lax.fori_loop(0, CHUNK // 16, _zwait, 0)
    plsc.subcore_barrier()

    base = wid * EPW

    def _idx_fetch(b, w):
        off = base + w * W
        pltpu.async_copy(src_hbm.at[pl.ds(off, W)], srcw[b], semi[b])
        pltpu.async_copy(dst_hbm.at[pl.ds(off, W)], dstw[b], semi[b])

    def _idx_wait(b, w):
        off = base + w * W
        pltpu.make_async_copy(src_hbm.at[pl.ds(off, W)], srcw[b],
                              semi[b]).wait()
        pltpu.make_async_copy(dst_hbm.at[pl.ds(off, W)], dstw[b],
                              semi[b]).wait()

    for b in range(2):
        _idx_fetch(b, b)
    _idx_wait(0, 0)
    pltpu.async_copy(xl_hbm.at[srcw[0]], rows[0], semg[0])

    def _block(i, carry):
        w0 = i * NBUF
        for b in range(NBUF):
            w = w0 + b
            bn = (b + 1) % NBUF
            pltpu.make_async_copy(
                xl_hbm.at[srcw[b]], rows[b], semg[b]).wait()

            @pl.when(w + 1 < WPW)
            def _():
                _idx_wait(bn, w + 1)
                pltpu.async_copy(xl_hbm.at[srcw[bn]], rows[bn], semg[bn])

            for g in range(W // 16):
                s16 = srcw[b][pl.ds(g * 16, 16)]
                d16 = dstw[b][pl.ds(g * 16, 16)]
                av = plsc.load_gather(asrc_v, [s16])
                bv = plsc.load_gather(adst_v, [d16])
                z = av + bv
                alpha = jnp.maximum(z, 0.2 * z)
                ex = jnp.exp(alpha - cv)
                exw[b][pl.ds(g * 16, 16)] = ex
                for r in range(16):
                    bc = jnp.broadcast_to(ex[r], (16,))
                    row = g * 16 + r
                    for j in range(D // 16):
                        rows[b][row, pl.ds(j * 16, 16)] = (
                            rows[b][row, pl.ds(j * 16, 16)] * bc)
            pltpu.async_copy(rows[b], acc.at[dstw[b]], sems[b], add=True)
            pltpu.async_copy(exw[b], den.at[dstw[b]], sems[b], add=True)

            bp = (b + 2) % NBUF

            @pl.when(jnp.logical_and(w >= 1, w + 2 < WPW))
            def _():
                pltpu.make_async_copy(
                    rows[bp], acc.at[dstw[bp]], sems[bp]).wait()
                pltpu.make_async_copy(
                    exw[bp], den.at[dstw[bp]], sems[bp]).wait()

            @pl.when(w + 2 < WPW)
            def _():
                _idx_fetch(bp, w + 2)
        return carry
    lax.fori_loop(0, WPW // NBUF, _block, 0)

    for b in range(NBUF):
        pltpu.make_async_copy(rows[b], acc.at[dstw[b]], sems[b]).wait()
        pltpu.make_async_copy(exw[b], den.at[dstw[b]], sems[b]).wait()
    plsc.subcore_barrier()
    pltpu.sync_copy(acc.at[pl.ds(s * CHUNK, CHUNK)],
                    acc_out.at[c, pl.ds(s * CHUNK, CHUNK)])
    pltpu.sync_copy(den.at[pl.ds(s * CHUNK, CHUNK)],
                    den_out.at[c, pl.ds(s * CHUNK, CHUNK)])


def kernel(x, edge_index, W1, att_src1, att_dst1, bias1,
           W2, att_src2, att_dst2, bias2):
    loop = jnp.arange(N, dtype=jnp.int32)
    pad = jnp.full((EPAD - E2,), N, jnp.int32)
    src_e = jnp.concatenate([edge_index[0].astype(jnp.int32), loop, pad])
    dst_e = jnp.concatenate([edge_index[1].astype(jnp.int32), loop, pad])
    xl1, as1, ad1, cv1 = _prep1(x, W1, att_src1, att_dst1)
    acc1, den1 = _gat_edge(xl1, src_e, dst_e,
                           as1.reshape(NPAD), ad1.reshape(NPAD),
                           cv1.reshape(16))
    xl2, as2, ad2, cv2 = _prep2(acc1, den1, bias1, W2, att_src2, att_dst2)
    acc2, den2 = _gat_edge(xl2, src_e, dst_e,
                           as2.reshape(NPAD), ad2.reshape(NPAD),
                           cv2.reshape(16))
    out = _final(acc2, den2, bias2)
    return out[:N]
